# baseline (speedup 1.0000x reference)
import functools

import jax
import jax.numpy as jnp
from jax import lax
from jax.experimental import pallas as pl
from jax.experimental.pallas import tpu as pltpu
from jax.experimental.pallas import tpu_sc as plsc

N = 6_400_000
NUM_CORES = 2
NUM_SUBCORES = 16
NW = NUM_CORES * NUM_SUBCORES
T = N // NW
S = 4_000
R = T // S
V = S // 16
U = 5
CH = 512
PAD = S + CH
CUT = 1_200_000
STG = CUT // NUM_SUBCORES
SCH = 3_000


def kernel(edge_weights, edge_index):
    mesh = plsc.VectorSubcoreMesh(
        core_axis_name="c", subcore_axis_name="s",
        num_cores=NUM_CORES, num_subcores=NUM_SUBCORES,
    )

    @functools.partial(
        pl.kernel,
        mesh=mesh,
        out_type=jax.ShapeDtypeStruct((N,), jnp.float32),
        scratch_types=(
            [pltpu.VMEM((S,), jnp.int32) for _ in range(2)]
            + [pltpu.VMEM((2 * PAD,), jnp.int32) for _ in range(2)]
            + [pltpu.VMEM((S,), jnp.float32) for _ in range(2)]
            + [pltpu.VMEM((2 * PAD,), jnp.float32) for _ in range(2)]
            + [pltpu.VMEM_SHARED((CUT,), jnp.float32)]
            + [pltpu.SemaphoreType.DMA for _ in range(8)]
        ),
        compiler_params=pltpu.CompilerParams(needs_layout_passes=False),
    )
    def gather_kernel(w_hbm, idx_hbm, zeros_hbm, out_hbm, *scratch):
        ix = scratch[0:2]
        idxb = scratch[2:4]
        pos = scratch[4:6]
        val = scratch[6:8]
        shared_w = scratch[8]
        s_in = scratch[9:11]
        s_sp = scratch[11:13]
        s_hb = scratch[13:15]
        s_o = scratch[15:17]

        sid = lax.axis_index("s")
        wid = sid * NUM_CORES + lax.axis_index("c")
        base = wid * T

        def stage(j, carry):
            off = sid * STG + j * SCH
            pltpu.sync_copy(w_hbm.at[pl.ds(off, SCH)],
                            val[0].at[pl.ds(0, SCH)])
            pltpu.sync_copy(val[0].at[pl.ds(0, SCH)],
                            shared_w.at[pl.ds(off, SCH)])
            return carry

        lax.fori_loop(0, STG // SCH, stage, 0)

        pltpu.sync_copy(zeros_hbm, idxb[0])
        pltpu.sync_copy(zeros_hbm, idxb[1])
        plsc.subcore_barrier()

        iota1 = lax.iota(jnp.int32, 16) + 1

        def in_copy(r, b):
            return pltpu.make_async_copy(
                idx_hbm.at[pl.ds(base + r * S, S)], ix[b], s_in[b])

        def out_copy(r, b):
            return pltpu.make_async_copy(
                pos[b], out_hbm.at[pl.ds(base + r * S, S)], s_o[b])

        def sp_chunk(b, j):
            return pltpu.make_async_copy(
                shared_w.at[idxb[b].at[pl.ds(j * CH, CH)]],
                val[b].at[pl.ds(j * CH, CH)], s_sp[b])

        def hb_chunk(b, j):
            return pltpu.make_async_copy(
                w_hbm.at[idxb[b].at[pl.ds(PAD + j * CH, CH)]],
                val[b].at[pl.ds(PAD + j * CH, CH)], s_hb[b])

        def compress(b):
            def cbody(i, carry):
                ospv, ohbv = carry
                for u in range(U):
                    v = i * U + u
                    x = ix[b][pl.ds(v * 16, 16)]
                    m = x < CUT
                    cnt = plsc.all_reduce_population_count(m)
                    cs = plsc.cumsum(jnp.where(m, 1, 0))
                    p = jnp.where(m, cs + (ospv - 1),
                                  (iota1 - cs) + (ohbv + (PAD - 1)))
                    plsc.store_scatter(idxb[b], [p], x)
                    pos[b][pl.ds(v * 16, 16)] = plsc.bitcast(p, jnp.float32)
                    ospv = ospv + cnt
                    ohbv = ohbv + (16 - cnt)
                return ospv, ohbv

            zero_v = jnp.zeros((16,), jnp.int32)
            ospv, ohbv = lax.fori_loop(0, V // U, cbody, (zero_v, zero_v))
            osp = ospv[0]
            ohb = ohbv[0]
            return (osp + (CH - 1)) // CH, (ohb + (CH - 1)) // CH

        def start_gathers(b, nsp, nhb):
            def sps(j, carry):
                sp_chunk(b, j).start()
                return carry

            def hbs(j, carry):
                hb_chunk(b, j).start()
                return carry

            lax.fori_loop(0, nsp, sps, 0)
            lax.fori_loop(0, nhb, hbs, 0)

        def drain_gathers(b, nsp, nhb):
            def spw(j, carry):
                sp_chunk(b, 0).wait()
                return carry

            def hbw(j, carry):
                hb_chunk(b, 0).wait()
                return carry

            lax.fori_loop(0, nsp, spw, 0)
            lax.fori_loop(0, nhb, hbw, 0)

        def expand(b):
            def ebody(i, carry):
                for u in range(U):
                    v = i * U + u
                    p = plsc.bitcast(pos[b][pl.ds(v * 16, 16)], jnp.int32)
                    pos[b][pl.ds(v * 16, 16)] = plsc.load_gather(val[b], [p])
                return carry

            lax.fori_loop(0, V // U, ebody, 0)

        def round_step(r, b, n_prev, *, wait_out, drain_prev, prefetch):
            in_copy(r, b).wait()
            if wait_out:
                out_copy(r - 2, b).wait()
            n_cur = compress(b)
            start_gathers(b, *n_cur)
            if drain_prev:
                pb = 1 - b
                drain_gathers(pb, *n_prev)
                expand(pb)
                out_copy(r - 1, pb).start()
            if prefetch:
                in_copy(r + 1, 1 - b).start()
            return n_cur

        in_copy(0, 0).start()
        in_copy(1, 1).start()
        n = round_step(0, 0, None, wait_out=False, drain_prev=False,
                       prefetch=False)
        n = round_step(1, 1, n, wait_out=False, drain_prev=True,
                       prefetch=True)

        def steady(i, carry):
            r0 = 2 + i * 2
            carry = round_step(r0, 0, carry, wait_out=True, drain_prev=True,
                               prefetch=True)
            carry = round_step(r0 + 1, 1, carry, wait_out=True,
                               drain_prev=True, prefetch=True)
            return carry

        n = lax.fori_loop(0, (R - 4) // 2, steady, n)

        n = round_step(R - 2, 0, n, wait_out=True, drain_prev=True,
                       prefetch=True)
        n = round_step(R - 1, 1, n, wait_out=True, drain_prev=True,
                       prefetch=False)

        drain_gathers(1, *n)
        expand(1)
        out_copy(R - 1, 1).start()
        out_copy(R - 2, 0).wait()
        out_copy(R - 1, 1).wait()

    zeros = jnp.zeros((2 * PAD,), jnp.int32)
    return gather_kernel(edge_weights, edge_index, zeros)


# speedup vs baseline: 9.1860x; 9.1860x over previous
import functools

import jax
import jax.numpy as jnp
from jax import lax
from jax.experimental import pallas as pl
from jax.experimental.pallas import tpu as pltpu
from jax.experimental.pallas import tpu_sc as plsc

N = 6_400_000
NUM_CORES = 2
NUM_SUBCORES = 16
NW = NUM_CORES * NUM_SUBCORES
T = N // NW
S = 10_000
R = T // S
NBUF = 4


def kernel(edge_weights, edge_index):
    mesh = plsc.VectorSubcoreMesh(
        core_axis_name="c", subcore_axis_name="s",
        num_cores=NUM_CORES, num_subcores=NUM_SUBCORES,
    )

    @functools.partial(
        pl.kernel,
        mesh=mesh,
        out_type=jax.ShapeDtypeStruct((N,), jnp.float32),
        scratch_types=(
            [pltpu.VMEM((S,), jnp.int32) for _ in range(NBUF)]
            + [pltpu.VMEM((S,), jnp.float32) for _ in range(NBUF)]
            + [pltpu.SemaphoreType.DMA for _ in range(4 * NBUF)]
        ),
    )
    def gather_kernel(w_hbm, idx_hbm, out_hbm, *scratch):
        ix = scratch[0:NBUF]
        vv = scratch[NBUF:2 * NBUF]
        s_in = scratch[2 * NBUF:3 * NBUF]
        s_g = scratch[3 * NBUF:4 * NBUF]
        s_o = scratch[4 * NBUF:5 * NBUF]
        s_g2 = scratch[5 * NBUF:6 * NBUF]

        wid = lax.axis_index("s") * NUM_CORES + lax.axis_index("c")
        base = wid * T

        H = S // 2

        class _GPair:
            def __init__(self, b):
                self.c1 = pltpu.make_async_copy(
                    w_hbm.at[ix[b].at[pl.ds(0, H)]],
                    vv[b].at[pl.ds(0, H)], s_g[b])
                self.c2 = pltpu.make_async_copy(
                    w_hbm.at[ix[b].at[pl.ds(H, H)]],
                    vv[b].at[pl.ds(H, H)], s_g2[b])

            def start(self):
                self.c1.start()
                self.c2.start()

            def wait(self):
                self.c1.wait()
                self.c2.wait()

        def g_copy(b):
            return _GPair(b)

        def out_copy(r, b):
            return pltpu.make_async_copy(
                vv[b], out_hbm.at[pl.ds(base + r * S, S)], s_o[b])

        def in_copy_d(r, b):
            return pltpu.make_async_copy(
                idx_hbm.at[pl.ds(base + r * S, S)], ix[b], s_in[b])

        def round_step(r, b, *, drain_out, wait_prev_g, prefetch):
            if drain_out:
                out_copy(r - NBUF, b).wait()
            in_copy_d(r, b).wait()
            g_copy(b).start()
            if wait_prev_g:
                pb = (b - 1) % NBUF
                g_copy(pb).wait()
                out_copy(r - 1, pb).start()
            if prefetch:
                nb = (b - 1) % NBUF
                in_copy_d(r + NBUF - 1, nb).start()

        for r in range(NBUF - 1):
            in_copy_d(r, r % NBUF).start()
        round_step(0, 0, drain_out=False, wait_prev_g=False, prefetch=True)
        for r in range(1, NBUF):
            round_step(r, r % NBUF, drain_out=False, wait_prev_g=True,
                       prefetch=True)

        def steady(i, carry):
            r0 = NBUF + i * NBUF
            for b in range(NBUF):
                round_step(r0 + b, b, drain_out=True, wait_prev_g=True,
                           prefetch=True)
            return carry

        n_steady = (R - NBUF) // NBUF - 1
        lax.fori_loop(0, n_steady, steady, 0)

        r0 = NBUF + n_steady * NBUF
        for b in range(NBUF):
            round_step(r0 + b, b, drain_out=True, wait_prev_g=True,
                       prefetch=(r0 + b + NBUF - 1 < R))

        r0 += NBUF
        for b in range(R - r0):
            round_step(r0 + b, b, drain_out=True, wait_prev_g=True,
                       prefetch=False)
        last_b = (R - 1) % NBUF
        g_copy(last_b).wait()
        out_copy(R - 1, last_b).start()
        for k in range(NBUF):
            out_copy(R - NBUF + k, (R - NBUF + k) % NBUF).wait()

    return gather_kernel(edge_weights, edge_index)
